# baseline (speedup 1.0000x reference)
import numpy as np
import jax
import jax.numpy as jnp
from jax.experimental import pallas as pl

_STRIDES = (8, 16, 32)
_IMG_W = 512
_ALL_ANCHORS = np.array(
    [[10, 13], [16, 30], [33, 23], [30, 61], [62, 45], [59, 119],
     [116, 90], [156, 198], [373, 326]], dtype=np.float32)
_ANCHOR_MASKS = ((0, 1, 2), (3, 4, 5), (6, 7, 8))
_NC = 20
_NCH = 5 + _NC
_NA = 3


def _decode_body(xs_ref, xm_ref, xl_ref, out_ref):
    eye = jnp.eye(_NCH, dtype=jnp.float32)
    row = 0
    for idx, ref in enumerate((xs_ref, xm_ref, xl_ref)):
        stride = float(_STRIDES[idx])
        ng = _IMG_W // _STRIDES[idx]
        n = ng * ng
        mask = _ANCHOR_MASKS[idx]
        x = ref[0]
        cidx = jax.lax.broadcasted_iota(jnp.int32, (_NCH, n), 1)
        gx = (cidx & (ng - 1)).astype(jnp.float32)
        gy = (cidx >> int(np.log2(ng))).astype(jnp.float32)
        ch = jax.lax.broadcasted_iota(jnp.int32, (_NCH, n), 0)
        for a in range(_NA):
            v = x[a * _NCH:(a + 1) * _NCH, :]
            sig = jax.nn.sigmoid(v)
            ex = jnp.exp(v)
            aw = float(_ALL_ANCHORS[mask[a], 0] / stride)
            ah = float(_ALL_ANCHORS[mask[a], 1] / stride)
            res = jnp.where(
                ch == 0, (sig + gx) * stride,
                jnp.where(
                    ch == 1, (sig + gy) * stride,
                    jnp.where(
                        ch == 2, ex * aw * stride,
                        jnp.where(ch == 3, ex * ah * stride, sig))))
            res_t = jax.lax.dot_general(
                res, eye, (((0,), (0,)), ((), ())),
                precision=jax.lax.Precision.HIGHEST,
                preferred_element_type=jnp.float32)
            out_ref[0, row:row + n, :] = res_t
            row += n


def kernel(xs, xm, xl):
    nb = xs.shape[0]
    xs2 = xs.reshape(nb, _NA * _NCH, 64 * 64)
    xm2 = xm.reshape(nb, _NA * _NCH, 32 * 32)
    xl2 = xl.reshape(nb, _NA * _NCH, 16 * 16)
    total = _NA * (64 * 64 + 32 * 32 + 16 * 16)
    out = pl.pallas_call(
        _decode_body,
        grid=(nb,),
        in_specs=[
            pl.BlockSpec((1, _NA * _NCH, 64 * 64), lambda b: (b, 0, 0)),
            pl.BlockSpec((1, _NA * _NCH, 32 * 32), lambda b: (b, 0, 0)),
            pl.BlockSpec((1, _NA * _NCH, 16 * 16), lambda b: (b, 0, 0)),
        ],
        out_specs=pl.BlockSpec((1, total, _NCH), lambda b: (b, 0, 0)),
        out_shape=jax.ShapeDtypeStruct((nb, total, _NCH), jnp.float32),
    )(xs2, xm2, xl2)
    return out


# speedup vs baseline: 1.4396x; 1.4396x over previous
import numpy as np
import jax
import jax.numpy as jnp
from jax.experimental import pallas as pl

_STRIDES = (8, 16, 32)
_IMG_W = 512
_ALL_ANCHORS = np.array(
    [[10, 13], [16, 30], [33, 23], [30, 61], [62, 45], [59, 119],
     [116, 90], [156, 198], [373, 326]], dtype=np.float32)
_ANCHOR_MASKS = ((0, 1, 2), (3, 4, 5), (6, 7, 8))
_NC = 20
_NCH = 5 + _NC
_NA = 3


def _decode_body(xs_ref, xm_ref, xl_ref, out_ref):
    row = 0
    for idx, ref in enumerate((xs_ref, xm_ref, xl_ref)):
        stride = float(_STRIDES[idx])
        ng = _IMG_W // _STRIDES[idx]
        n = ng * ng
        mask = _ANCHOR_MASKS[idx]
        x = ref[0]
        sig = jax.nn.sigmoid(x)
        ch8 = jax.lax.broadcasted_iota(jnp.int32, (8, n), 0)
        pix = jax.lax.broadcasted_iota(jnp.int32, (8, n), 1)
        gx = (pix & (ng - 1)).astype(jnp.float32)
        gy = (pix >> int(np.log2(ng))).astype(jnp.float32)
        pieces = []
        prev = 0
        for a in range(_NA):
            t0 = (25 * a) // 8 * 8
            o = 25 * a - t0
            xt = x[t0:t0 + 8, :]
            st = sig[t0:t0 + 8, :]
            et = jnp.exp(xt)
            aw = float(_ALL_ANCHORS[mask[a], 0] / stride)
            ah = float(_ALL_ANCHORS[mask[a], 1] / stride)
            fix = jnp.where(
                ch8 == o, (st + gx) * stride,
                jnp.where(
                    ch8 == o + 1, (st + gy) * stride,
                    jnp.where(
                        ch8 == o + 2, et * aw * stride,
                        jnp.where(ch8 == o + 3, et * ah * stride, st))))
            if t0 > prev:
                pieces.append(sig[prev:t0, :])
            pieces.append(fix)
            prev = t0 + 8
        pieces.append(sig[prev:_NA * _NCH, :])
        res = jnp.concatenate(pieces, axis=0)
        t = res.T
        for a in range(_NA):
            out_ref[0, row:row + n, :] = t[:, 25 * a:25 * a + 25]
            row += n


def kernel(xs, xm, xl):
    nb = xs.shape[0]
    xs2 = xs.reshape(nb, _NA * _NCH, 64 * 64)
    xm2 = xm.reshape(nb, _NA * _NCH, 32 * 32)
    xl2 = xl.reshape(nb, _NA * _NCH, 16 * 16)
    total = _NA * (64 * 64 + 32 * 32 + 16 * 16)
    out = pl.pallas_call(
        _decode_body,
        grid=(nb,),
        in_specs=[
            pl.BlockSpec((1, _NA * _NCH, 64 * 64), lambda b: (b, 0, 0)),
            pl.BlockSpec((1, _NA * _NCH, 32 * 32), lambda b: (b, 0, 0)),
            pl.BlockSpec((1, _NA * _NCH, 16 * 16), lambda b: (b, 0, 0)),
        ],
        out_specs=pl.BlockSpec((1, total, _NCH), lambda b: (b, 0, 0)),
        out_shape=jax.ShapeDtypeStruct((nb, total, _NCH), jnp.float32),
    )(xs2, xm2, xl2)
    return out
